# Initial kernel scaffold; baseline (speedup 1.0000x reference)
import jax
import jax.numpy as jnp
from jax.experimental import pallas as pl


def kernel(LE_1, indices_prev, indices_1, l_tuples, multiplicities_t):
    raise NotImplementedError("write your pallas kernel here")


# speedup vs baseline: 2.3604x; 2.3604x over previous
import functools

import jax
import jax.numpy as jnp
from jax.experimental import pallas as pl
from jax.experimental.pallas import tpu as pltpu

_M = 7
_N = 16
_A = 8192
_AT = 2048


def _tc_body(rows_a_ref, rows_b_ref, mult_ref, a_ref, b_ref, o_ref):
    tk = pl.program_id(0)
    m = mult_ref[tk]
    bm = b_ref[0] * m
    for i in range(_N):
        a_row = a_ref[0, pl.ds(i, 1), :]
        o_ref[0, i] = jnp.broadcast_to(a_row, (_N, _AT)) * bm


def kernel(LE_1, indices_prev, indices_1, l_tuples, multiplicities_t):
    T, K = indices_prev.shape
    TK = T * K
    rows_a = (l_tuples[:, 0][:, None] * _M + indices_prev).reshape(-1)
    rows_b = (l_tuples[:, 1][:, None] * _M + indices_1).reshape(-1)
    rows_a = rows_a.astype(jnp.int32)
    rows_b = rows_b.astype(jnp.int32)
    mult = multiplicities_t.reshape(-1)
    le_flat = LE_1.reshape(-1, _N, _A)

    grid = (TK, _A // _AT)
    out = pl.pallas_call(
        _tc_body,
        grid_spec=pltpu.PrefetchScalarGridSpec(
            num_scalar_prefetch=3,
            grid=grid,
            in_specs=[
                pl.BlockSpec((1, _N, _AT), lambda tk, ab, ra, rb, mu: (ra[tk], 0, ab)),
                pl.BlockSpec((1, _N, _AT), lambda tk, ab, ra, rb, mu: (rb[tk], 0, ab)),
            ],
            out_specs=pl.BlockSpec(
                (1, _N, _N, _AT), lambda tk, ab, ra, rb, mu: (tk, 0, 0, ab)
            ),
        ),
        out_shape=jax.ShapeDtypeStruct((TK, _N, _N, _A), jnp.float32),
        compiler_params=pltpu.CompilerParams(
            dimension_semantics=("arbitrary", "arbitrary"),
        ),
    )(rows_a, rows_b, mult, le_flat, le_flat)
    return out.reshape(T, K, _N * _N, _A)


# speedup vs baseline: 3.0775x; 1.3038x over previous
import functools

import jax
import jax.numpy as jnp
from jax.experimental import pallas as pl
from jax.experimental.pallas import tpu as pltpu

_M = 7
_N = 16
_A = 8192
_AT = 4096


def _tc_body(rows_a_ref, rows_b_ref, mult_ref, a_ref, b_ref, o_ref):
    tk = pl.program_id(0)
    m = mult_ref[tk]
    bm = b_ref[0] * m
    for i in range(_N):
        a_row = a_ref[0, pl.ds(i, 1), :]
        o_ref[0, i] = jnp.broadcast_to(a_row, (_N, _AT)) * bm


def kernel(LE_1, indices_prev, indices_1, l_tuples, multiplicities_t):
    T, K = indices_prev.shape
    TK = T * K
    rows_a = (l_tuples[:, 0][:, None] * _M + indices_prev).reshape(-1)
    rows_b = (l_tuples[:, 1][:, None] * _M + indices_1).reshape(-1)
    rows_a = rows_a.astype(jnp.int32)
    rows_b = rows_b.astype(jnp.int32)
    mult = multiplicities_t.reshape(-1)
    le_flat = LE_1.reshape(-1, _N, _A)

    grid = (TK, _A // _AT)
    out = pl.pallas_call(
        _tc_body,
        grid_spec=pltpu.PrefetchScalarGridSpec(
            num_scalar_prefetch=3,
            grid=grid,
            in_specs=[
                pl.BlockSpec((1, _N, _AT), lambda tk, ab, ra, rb, mu: (ra[tk], 0, ab)),
                pl.BlockSpec((1, _N, _AT), lambda tk, ab, ra, rb, mu: (rb[tk], 0, ab)),
            ],
            out_specs=pl.BlockSpec(
                (1, _N, _N, _AT), lambda tk, ab, ra, rb, mu: (tk, 0, 0, ab)
            ),
        ),
        out_shape=jax.ShapeDtypeStruct((TK, _N, _N, _A), jnp.float32),
        compiler_params=pltpu.CompilerParams(
            dimension_semantics=("arbitrary", "arbitrary"),
        ),
    )(rows_a, rows_b, mult, le_flat, le_flat)
    return out.reshape(T, K, _N * _N, _A)


# speedup vs baseline: 3.4669x; 1.1265x over previous
import functools

import jax
import jax.numpy as jnp
from jax.experimental import pallas as pl
from jax.experimental.pallas import tpu as pltpu

_M = 7
_N = 16
_A = 8192
_AT = 8192


def _tc_body(rows_a_ref, rows_b_ref, mult_ref, a_ref, b_ref, o_ref):
    tk = pl.program_id(0)
    m = mult_ref[tk]
    bm = b_ref[0] * m
    for i in range(_N):
        a_row = a_ref[0, pl.ds(i, 1), :]
        o_ref[0, i] = jnp.broadcast_to(a_row, (_N, _AT)) * bm


def kernel(LE_1, indices_prev, indices_1, l_tuples, multiplicities_t):
    T, K = indices_prev.shape
    TK = T * K
    rows_a = (l_tuples[:, 0][:, None] * _M + indices_prev).reshape(-1)
    rows_b = (l_tuples[:, 1][:, None] * _M + indices_1).reshape(-1)
    rows_a = rows_a.astype(jnp.int32)
    rows_b = rows_b.astype(jnp.int32)
    mult = multiplicities_t.reshape(-1)
    le_flat = LE_1.reshape(-1, _N, _A)

    grid = (TK, _A // _AT)
    out = pl.pallas_call(
        _tc_body,
        grid_spec=pltpu.PrefetchScalarGridSpec(
            num_scalar_prefetch=3,
            grid=grid,
            in_specs=[
                pl.BlockSpec((1, _N, _AT), lambda tk, ab, ra, rb, mu: (ra[tk], 0, ab)),
                pl.BlockSpec((1, _N, _AT), lambda tk, ab, ra, rb, mu: (rb[tk], 0, ab)),
            ],
            out_specs=pl.BlockSpec(
                (1, _N, _N, _AT), lambda tk, ab, ra, rb, mu: (tk, 0, 0, ab)
            ),
        ),
        out_shape=jax.ShapeDtypeStruct((TK, _N, _N, _A), jnp.float32),
        compiler_params=pltpu.CompilerParams(
            dimension_semantics=("arbitrary", "arbitrary"),
        ),
    )(rows_a, rows_b, mult, le_flat, le_flat)
    return out.reshape(T, K, _N * _N, _A)
